# Initial kernel scaffold; baseline (speedup 1.0000x reference)
import jax
import jax.numpy as jnp
from jax.experimental import pallas as pl


def kernel(features, edge_index, W1, b1, W2, b2):
    raise NotImplementedError("write your pallas kernel here")


# speedup vs baseline: 2.4742x; 2.4742x over previous
import functools

import jax
import jax.numpy as jnp
from jax import lax
from jax.experimental import pallas as pl
from jax.experimental.pallas import tpu as pltpu
from jax.experimental.pallas import tpu_sc as plsc

N = 10000
E = 160000
D = 256
H = 128
NT = 16
C = 128
NCH = 80
E_PAD = NT * C * NCH
ACC_ROWS = 10240
ZROWS = 640
RBLK = 400
GRID = N // RBLK


def _seg_mean(with_deg):
    outs = [jax.ShapeDtypeStruct((2 * ACC_ROWS, H), jnp.float32)]
    if with_deg:
        outs.append(jax.ShapeDtypeStruct((2 * ACC_ROWS, H), jnp.float32))
    scratch = [
        pltpu.VMEM((1, C), jnp.int32),
        pltpu.VMEM((C,), jnp.int32),
        pltpu.VMEM((1, C), jnp.int32),
        pltpu.VMEM((C, H), jnp.float32),
        pltpu.VMEM_SHARED((ACC_ROWS, H), jnp.float32),
        pltpu.SemaphoreType.DMA,
    ]
    mesh = plsc.VectorSubcoreMesh(core_axis_name="c", subcore_axis_name="s",
                                  num_cores=2, num_subcores=NT)

    @functools.partial(pl.kernel, out_type=outs, mesh=mesh,
                       scratch_types=scratch)
    def k(y2n, srcs2, dsts, rowidx, zrow, ones_in, *rest):
        if with_deg:
            agg_out, deg_out, idx_v, src_v, dst_v, rows_v, acc, sem = rest
        else:
            agg_out, idx_v, src_v, dst_v, rows_v, acc, sem = rest
        c = lax.axis_index("c")
        s = lax.axis_index("s")

        def zero_slab():
            pltpu.sync_copy(zrow, rows_v)
            for j in range(ZROWS // C):
                pltpu.sync_copy(rowidx.at[pl.ds(s * ZROWS + j * C, C)],
                                idx_v.at[0])
                pltpu.sync_copy(rows_v, acc.at[idx_v.at[0]])

        def read_slab(out_ref):
            for j in range(ZROWS // C):
                pltpu.sync_copy(rowidx.at[pl.ds(s * ZROWS + j * C, C)],
                                idx_v.at[0])
                orows = pl.ds(c * ACC_ROWS + s * ZROWS + j * C, C)
                pltpu.async_copy(acc.at[idx_v.at[0]], rows_v, sem).wait()
                pltpu.sync_copy(rows_v, out_ref.at[orows])

        if with_deg:
            zero_slab()
            pltpu.sync_copy(ones_in, rows_v)
            plsc.subcore_barrier()

            def dstep(kk, carry):
                pltpu.sync_copy(dsts.at[pl.ds((s * NCH + kk) * C, C)],
                                dst_v.at[0])
                pltpu.sync_copy(rows_v, acc.at[dst_v.at[0]], add=True)
                return carry

            lax.fori_loop(0, NCH, dstep, 0)
            plsc.subcore_barrier()
            read_slab(deg_out)
            plsc.subcore_barrier()

        zero_slab()
        plsc.subcore_barrier()

        def step(kk, carry):
            base = c * E_PAD + (s * NCH + kk) * C
            pltpu.sync_copy(srcs2.at[pl.ds(base, C)], src_v)
            pltpu.sync_copy(dsts.at[pl.ds((s * NCH + kk) * C, C)],
                            dst_v.at[0])
            pltpu.async_copy(y2n.at[src_v], rows_v, sem).wait()
            pltpu.sync_copy(rows_v, acc.at[dst_v.at[0]], add=True)
            return carry

        lax.fori_loop(0, NCH, step, 0)
        plsc.subcore_barrier()
        read_slab(agg_out)

    return k


def _mm(x, w):
    def body(x_ref, w_ref, o_ref):
        r = jnp.dot(x_ref[...], w_ref[...], preferred_element_type=jnp.float32)
        o_ref[0] = r[:, :H]
        o_ref[1] = r[:, H:]

    return pl.pallas_call(
        body, grid=(GRID,),
        in_specs=[pl.BlockSpec((RBLK, D), lambda i: (i, 0)),
                  pl.BlockSpec((D, D), lambda i: (0, 0))],
        out_specs=pl.BlockSpec((2, RBLK, H), lambda i: (0, i, 0)),
        out_shape=jax.ShapeDtypeStruct((2, N, H), jnp.float32),
    )(x, w)


def _mid(y_st, agg_st, deg_st, b, w):
    def body(y_ref, a_ref, deg_ref, b_ref, w_ref, o_ref):
        inv = 1.0 / jnp.maximum(deg_ref[0][:, 0:1], 1.0)
        xlo = y_ref[0] + a_ref[0] * inv
        xhi = y_ref[1] + a_ref[1] * inv
        xx = jnp.concatenate([xlo, xhi], axis=1) + b_ref[...]
        xx = jnp.maximum(xx, 0.0)
        r = jnp.dot(xx, w_ref[...], preferred_element_type=jnp.float32)
        o_ref[0] = r[:, :H]
        o_ref[1] = r[:, H:]

    return pl.pallas_call(
        body, grid=(GRID,),
        in_specs=[pl.BlockSpec((2, RBLK, H), lambda i: (0, i, 0)),
                  pl.BlockSpec((2, RBLK, H), lambda i: (0, i, 0)),
                  pl.BlockSpec((1, RBLK, H), lambda i: (0, i, 0)),
                  pl.BlockSpec((1, D), lambda i: (0, 0)),
                  pl.BlockSpec((D, D), lambda i: (0, 0))],
        out_specs=pl.BlockSpec((2, RBLK, H), lambda i: (0, i, 0)),
        out_shape=jax.ShapeDtypeStruct((2, N, H), jnp.float32),
    )(y_st, agg_st, deg_st, b, w)


def _fin(y_st, agg_st, deg_st, b):
    def body(y_ref, a_ref, deg_ref, b_ref, o_ref):
        inv = 1.0 / jnp.maximum(deg_ref[0][:, 0:1], 1.0)
        xlo = y_ref[0] + a_ref[0] * inv
        xhi = y_ref[1] + a_ref[1] * inv
        o_ref[...] = jnp.concatenate([xlo, xhi], axis=1) + b_ref[...]

    return pl.pallas_call(
        body, grid=(GRID,),
        in_specs=[pl.BlockSpec((2, RBLK, H), lambda i: (0, i, 0)),
                  pl.BlockSpec((2, RBLK, H), lambda i: (0, i, 0)),
                  pl.BlockSpec((1, RBLK, H), lambda i: (0, i, 0)),
                  pl.BlockSpec((1, D), lambda i: (0, 0))],
        out_specs=pl.BlockSpec((RBLK, D), lambda i: (i, 0)),
        out_shape=jax.ShapeDtypeStruct((N, D), jnp.float32),
    )(y_st, agg_st, deg_st, b)


def kernel(features, edge_index, W1, b1, W2, b2):
    x = features.astype(jnp.float32)
    ei = edge_index.astype(jnp.int32)
    pad = E_PAD - E
    srcs = jnp.concatenate([ei[0], jnp.zeros((pad,), jnp.int32)])
    srcs2 = jnp.concatenate([srcs, srcs + N])
    dsts = jnp.concatenate([ei[1], jnp.full((pad,), N, jnp.int32)])
    rowidx = jnp.arange(ACC_ROWS, dtype=jnp.int32)
    zrow = jnp.zeros((C, H), jnp.float32)
    ones = jnp.ones((C, H), jnp.float32)
    b1r = b1.reshape(1, D)
    b2r = b2.reshape(1, D)

    y1 = _mm(x, W1)
    a1f, degf = _seg_mean(True)(y1.reshape(2 * N, H), srcs2, dsts,
                                rowidx, zrow, ones)
    a1 = a1f.reshape(2, ACC_ROWS, H)
    deg = degf.reshape(2, ACC_ROWS, H)
    y2 = _mid(y1, a1, deg, b1r, W2)
    a2f, = _seg_mean(False)(y2.reshape(2 * N, H), srcs2, dsts,
                            rowidx, zrow, ones)
    a2 = a2f.reshape(2, ACC_ROWS, H)
    return _fin(y2, a2, deg, b2r)


# speedup vs baseline: 3.0277x; 1.2237x over previous
import functools

import jax
import jax.numpy as jnp
from jax import lax
from jax.experimental import pallas as pl
from jax.experimental.pallas import tpu as pltpu
from jax.experimental.pallas import tpu_sc as plsc

N = 10000
E = 160000
D = 256
H = 128
NT = 16
C = 128
NCH = 80
E_PAD = NT * C * NCH
ACC_ROWS = 10240
ZROWS = 640
RBLK = 400
GRID = N // RBLK


def _seg_mean(with_deg):
    outs = [jax.ShapeDtypeStruct((2 * ACC_ROWS, H), jnp.float32)]
    if with_deg:
        outs.append(jax.ShapeDtypeStruct((2 * ACC_ROWS, H), jnp.float32))
    scratch = [
        pltpu.VMEM((1, C), jnp.int32),
        pltpu.VMEM((C,), jnp.int32),
        pltpu.VMEM((C,), jnp.int32),
        pltpu.VMEM((2, C), jnp.int32),
        pltpu.VMEM((C, H), jnp.float32),
        pltpu.VMEM((C, H), jnp.float32),
        pltpu.VMEM_SHARED((ACC_ROWS, H), jnp.float32),
        pltpu.SemaphoreType.DMA,
        pltpu.SemaphoreType.DMA,
    ]
    mesh = plsc.VectorSubcoreMesh(core_axis_name="c", subcore_axis_name="s",
                                  num_cores=2, num_subcores=NT)

    @functools.partial(pl.kernel, out_type=outs, mesh=mesh,
                       scratch_types=scratch)
    def k(y2n, srcs2, dsts, rowidx, zrow, ones_in, *rest):
        if with_deg:
            (agg_out, deg_out, idx_v, src_v0, src_v1, dst_v,
             rows_v0, rows_v1, acc, sem0, sem1) = rest
        else:
            (agg_out, idx_v, src_v0, src_v1, dst_v,
             rows_v0, rows_v1, acc, sem0, sem1) = rest
        rows_v = rows_v0
        c = lax.axis_index("c")
        s = lax.axis_index("s")

        def zero_slab():
            pltpu.sync_copy(zrow, rows_v)
            for j in range(ZROWS // C):
                pltpu.sync_copy(rowidx.at[pl.ds(s * ZROWS + j * C, C)],
                                idx_v.at[0])
                pltpu.sync_copy(rows_v, acc.at[idx_v.at[0]])

        def read_slab(out_ref):
            for j in range(ZROWS // C):
                pltpu.sync_copy(rowidx.at[pl.ds(s * ZROWS + j * C, C)],
                                idx_v.at[0])
                orows = pl.ds(c * ACC_ROWS + s * ZROWS + j * C, C)
                pltpu.async_copy(acc.at[idx_v.at[0]], rows_v, sem0).wait()
                pltpu.sync_copy(rows_v, out_ref.at[orows])

        if with_deg:
            zero_slab()
            pltpu.sync_copy(ones_in, rows_v)
            plsc.subcore_barrier()

            def dstep(kk, carry):
                pltpu.sync_copy(dsts.at[pl.ds((s * NCH + kk) * C, C)],
                                dst_v.at[0])
                pltpu.sync_copy(rows_v, acc.at[dst_v.at[0]], add=True)
                return carry

            lax.fori_loop(0, NCH, dstep, 0, unroll=2)
            plsc.subcore_barrier()
            read_slab(deg_out)
            plsc.subcore_barrier()

        zero_slab()
        plsc.subcore_barrier()

        def step2(t, carry):
            k0 = 2 * t
            base0 = c * E_PAD + (s * NCH + k0) * C
            pltpu.sync_copy(srcs2.at[pl.ds(base0, C)], src_v0)
            g0 = pltpu.async_copy(y2n.at[src_v0], rows_v0, sem0)
            pltpu.sync_copy(dsts.at[pl.ds((s * NCH + k0) * C, C)],
                            dst_v.at[0])
            pltpu.sync_copy(srcs2.at[pl.ds(base0 + C, C)], src_v1)
            g1 = pltpu.async_copy(y2n.at[src_v1], rows_v1, sem1)
            pltpu.sync_copy(dsts.at[pl.ds((s * NCH + k0 + 1) * C, C)],
                            dst_v.at[1])
            g0.wait()
            pltpu.sync_copy(rows_v0, acc.at[dst_v.at[0]], add=True)
            g1.wait()
            pltpu.sync_copy(rows_v1, acc.at[dst_v.at[1]], add=True)
            return carry

        lax.fori_loop(0, NCH // 2, step2, 0)
        plsc.subcore_barrier()
        read_slab(agg_out)

    return k


def _mm(x, w):
    def body(x_ref, w_ref, o_ref):
        r = jnp.dot(x_ref[...], w_ref[...], preferred_element_type=jnp.float32)
        o_ref[0] = r[:, :H]
        o_ref[1] = r[:, H:]

    return pl.pallas_call(
        body, grid=(GRID,),
        in_specs=[pl.BlockSpec((RBLK, D), lambda i: (i, 0)),
                  pl.BlockSpec((D, D), lambda i: (0, 0))],
        out_specs=pl.BlockSpec((2, RBLK, H), lambda i: (0, i, 0)),
        out_shape=jax.ShapeDtypeStruct((2, N, H), jnp.float32),
    )(x, w)


def _mid(y_st, agg_st, deg_st, b, w):
    def body(y_ref, a_ref, deg_ref, b_ref, w_ref, o_ref):
        inv = 1.0 / jnp.maximum(deg_ref[0][:, 0:1], 1.0)
        xlo = y_ref[0] + a_ref[0] * inv
        xhi = y_ref[1] + a_ref[1] * inv
        xx = jnp.concatenate([xlo, xhi], axis=1) + b_ref[...]
        xx = jnp.maximum(xx, 0.0)
        r = jnp.dot(xx, w_ref[...], preferred_element_type=jnp.float32)
        o_ref[0] = r[:, :H]
        o_ref[1] = r[:, H:]

    return pl.pallas_call(
        body, grid=(GRID,),
        in_specs=[pl.BlockSpec((2, RBLK, H), lambda i: (0, i, 0)),
                  pl.BlockSpec((2, RBLK, H), lambda i: (0, i, 0)),
                  pl.BlockSpec((1, RBLK, H), lambda i: (0, i, 0)),
                  pl.BlockSpec((1, D), lambda i: (0, 0)),
                  pl.BlockSpec((D, D), lambda i: (0, 0))],
        out_specs=pl.BlockSpec((2, RBLK, H), lambda i: (0, i, 0)),
        out_shape=jax.ShapeDtypeStruct((2, N, H), jnp.float32),
    )(y_st, agg_st, deg_st, b, w)


def _fin(y_st, agg_st, deg_st, b):
    def body(y_ref, a_ref, deg_ref, b_ref, o_ref):
        inv = 1.0 / jnp.maximum(deg_ref[0][:, 0:1], 1.0)
        xlo = y_ref[0] + a_ref[0] * inv
        xhi = y_ref[1] + a_ref[1] * inv
        o_ref[...] = jnp.concatenate([xlo, xhi], axis=1) + b_ref[...]

    return pl.pallas_call(
        body, grid=(GRID,),
        in_specs=[pl.BlockSpec((2, RBLK, H), lambda i: (0, i, 0)),
                  pl.BlockSpec((2, RBLK, H), lambda i: (0, i, 0)),
                  pl.BlockSpec((1, RBLK, H), lambda i: (0, i, 0)),
                  pl.BlockSpec((1, D), lambda i: (0, 0))],
        out_specs=pl.BlockSpec((RBLK, D), lambda i: (i, 0)),
        out_shape=jax.ShapeDtypeStruct((N, D), jnp.float32),
    )(y_st, agg_st, deg_st, b)


def kernel(features, edge_index, W1, b1, W2, b2):
    x = features.astype(jnp.float32)
    ei = edge_index.astype(jnp.int32)
    pad = E_PAD - E
    srcs = jnp.concatenate([ei[0], jnp.zeros((pad,), jnp.int32)])
    srcs2 = jnp.concatenate([srcs, srcs + N])
    dsts = jnp.concatenate([ei[1], jnp.full((pad,), N, jnp.int32)])
    rowidx = jnp.arange(ACC_ROWS, dtype=jnp.int32)
    zrow = jnp.zeros((C, H), jnp.float32)
    ones = jnp.ones((C, H), jnp.float32)
    b1r = b1.reshape(1, D)
    b2r = b2.reshape(1, D)

    y1 = _mm(x, W1)
    a1f, degf = _seg_mean(True)(y1.reshape(2 * N, H), srcs2, dsts,
                                rowidx, zrow, ones)
    a1 = a1f.reshape(2, ACC_ROWS, H)
    deg = degf.reshape(2, ACC_ROWS, H)
    y2 = _mid(y1, a1, deg, b1r, W2)
    a2f, = _seg_mean(False)(y2.reshape(2 * N, H), srcs2, dsts,
                            rowidx, zrow, ones)
    a2 = a2f.reshape(2, ACC_ROWS, H)
    return _fin(y2, a2, deg, b2r)


# speedup vs baseline: 3.3996x; 1.1228x over previous
import functools

import jax
import jax.numpy as jnp
from jax import lax
from jax.experimental import pallas as pl
from jax.experimental.pallas import tpu as pltpu
from jax.experimental.pallas import tpu_sc as plsc

N = 10000
E = 160000
D = 256
H = 128
NT = 16
C = 128
NCH = 80
E_PAD = NT * C * NCH
ACC_ROWS = 10240
ZROWS = 640
BB = 8
RBLK = 400
GRID = N // RBLK


def _seg_mean(with_deg):
    outs = [jax.ShapeDtypeStruct((2 * ACC_ROWS, H), jnp.float32)]
    if with_deg:
        outs.append(jax.ShapeDtypeStruct((2 * ACC_ROWS, H), jnp.float32))
    scratch = [
        pltpu.VMEM((ZROWS // C, C), jnp.int32),
        pltpu.VMEM((BB * C,), jnp.int32),
        pltpu.VMEM((BB, C), jnp.int32),
        pltpu.VMEM((C, H), jnp.float32),
        pltpu.VMEM((C, H), jnp.float32),
        pltpu.VMEM_SHARED((ACC_ROWS, H), jnp.float32),
        pltpu.SemaphoreType.DMA,
        pltpu.SemaphoreType.DMA,
    ]
    mesh = plsc.VectorSubcoreMesh(core_axis_name="c", subcore_axis_name="s",
                                  num_cores=2, num_subcores=NT)

    @functools.partial(pl.kernel, out_type=outs, mesh=mesh,
                       scratch_types=scratch)
    def k(y2n, srcs2, dsts2, rowidx2, zrow, ones_in, *rest):
        if with_deg:
            (agg_out, deg_out, idx_v, src_b, dst_b,
             rows_v0, rows_v1, acc, sem0, sem1) = rest
        else:
            (agg_out, idx_v, src_b, dst_b,
             rows_v0, rows_v1, acc, sem0, sem1) = rest
        rows = (rows_v0, rows_v1)
        sems = (sem0, sem1)
        c = lax.axis_index("c")
        s = lax.axis_index("s")
        nslab = ZROWS // C

        def load_slab_idx():
            pltpu.sync_copy(rowidx2.at[s], idx_v)

        def zero_slab():
            pltpu.sync_copy(zrow, rows_v0)
            for j in range(nslab):
                pltpu.sync_copy(rows_v0, acc.at[idx_v.at[j]])

        def read_slab(out_ref):
            g = pltpu.async_copy(acc.at[idx_v.at[0]], rows_v0, sem0)
            for j in range(nslab):
                if j + 1 < nslab:
                    gn = pltpu.async_copy(acc.at[idx_v.at[j + 1]],
                                          rows[(j + 1) % 2], sems[(j + 1) % 2])
                g.wait()
                orows = pl.ds(c * ACC_ROWS + s * ZROWS + j * C, C)
                pltpu.sync_copy(rows[j % 2], out_ref.at[orows])
                if j + 1 < nslab:
                    g = gn

        load_slab_idx()
        if with_deg:
            zero_slab()
            pltpu.sync_copy(ones_in, rows_v1)
            plsc.subcore_barrier()

            def dstep(t, carry):
                pltpu.sync_copy(dsts2.at[pl.ds(s * NCH + t * BB, BB)], dst_b)
                for j in range(BB):
                    pltpu.sync_copy(rows_v1, acc.at[dst_b.at[j]], add=True)
                return carry

            lax.fori_loop(0, NCH // BB, dstep, 0)
            plsc.subcore_barrier()
            read_slab(deg_out)
            plsc.subcore_barrier()

        zero_slab()
        plsc.subcore_barrier()

        def step(t, carry):
            base = c * E_PAD + (s * NCH + t * BB) * C
            pltpu.sync_copy(srcs2.at[pl.ds(base, BB * C)], src_b)
            pltpu.sync_copy(dsts2.at[pl.ds(s * NCH + t * BB, BB)], dst_b)
            g = pltpu.async_copy(y2n.at[src_b.at[pl.ds(0, C)]],
                                 rows_v0, sem0)
            for j in range(BB):
                if j + 1 < BB:
                    gn = pltpu.async_copy(
                        y2n.at[src_b.at[pl.ds((j + 1) * C, C)]],
                        rows[(j + 1) % 2], sems[(j + 1) % 2])
                g.wait()
                pltpu.sync_copy(rows[j % 2], acc.at[dst_b.at[j]], add=True)
                if j + 1 < BB:
                    g = gn
            return carry

        lax.fori_loop(0, NCH // BB, step, 0)
        plsc.subcore_barrier()
        read_slab(agg_out)

    return k


def _mm(x, w):
    def body(x_ref, w_ref, o_ref):
        r = jnp.dot(x_ref[...], w_ref[...], preferred_element_type=jnp.float32)
        o_ref[0] = r[:, :H]
        o_ref[1] = r[:, H:]

    return pl.pallas_call(
        body, grid=(GRID,),
        in_specs=[pl.BlockSpec((RBLK, D), lambda i: (i, 0)),
                  pl.BlockSpec((D, D), lambda i: (0, 0))],
        out_specs=pl.BlockSpec((2, RBLK, H), lambda i: (0, i, 0)),
        out_shape=jax.ShapeDtypeStruct((2, N, H), jnp.float32),
    )(x, w)


def _mid(y_st, agg_st, deg_st, b, w):
    def body(y_ref, a_ref, deg_ref, b_ref, w_ref, o_ref):
        inv = 1.0 / jnp.maximum(deg_ref[0][:, 0:1], 1.0)
        xlo = y_ref[0] + a_ref[0] * inv
        xhi = y_ref[1] + a_ref[1] * inv
        xx = jnp.concatenate([xlo, xhi], axis=1) + b_ref[...]
        xx = jnp.maximum(xx, 0.0)
        r = jnp.dot(xx, w_ref[...], preferred_element_type=jnp.float32)
        o_ref[0] = r[:, :H]
        o_ref[1] = r[:, H:]

    return pl.pallas_call(
        body, grid=(GRID,),
        in_specs=[pl.BlockSpec((2, RBLK, H), lambda i: (0, i, 0)),
                  pl.BlockSpec((2, RBLK, H), lambda i: (0, i, 0)),
                  pl.BlockSpec((1, RBLK, H), lambda i: (0, i, 0)),
                  pl.BlockSpec((1, D), lambda i: (0, 0)),
                  pl.BlockSpec((D, D), lambda i: (0, 0))],
        out_specs=pl.BlockSpec((2, RBLK, H), lambda i: (0, i, 0)),
        out_shape=jax.ShapeDtypeStruct((2, N, H), jnp.float32),
    )(y_st, agg_st, deg_st, b, w)


def _fin(y_st, agg_st, deg_st, b):
    def body(y_ref, a_ref, deg_ref, b_ref, o_ref):
        inv = 1.0 / jnp.maximum(deg_ref[0][:, 0:1], 1.0)
        xlo = y_ref[0] + a_ref[0] * inv
        xhi = y_ref[1] + a_ref[1] * inv
        o_ref[...] = jnp.concatenate([xlo, xhi], axis=1) + b_ref[...]

    return pl.pallas_call(
        body, grid=(GRID,),
        in_specs=[pl.BlockSpec((2, RBLK, H), lambda i: (0, i, 0)),
                  pl.BlockSpec((2, RBLK, H), lambda i: (0, i, 0)),
                  pl.BlockSpec((1, RBLK, H), lambda i: (0, i, 0)),
                  pl.BlockSpec((1, D), lambda i: (0, 0))],
        out_specs=pl.BlockSpec((RBLK, D), lambda i: (i, 0)),
        out_shape=jax.ShapeDtypeStruct((N, D), jnp.float32),
    )(y_st, agg_st, deg_st, b)


def kernel(features, edge_index, W1, b1, W2, b2):
    x = features.astype(jnp.float32)
    ei = edge_index.astype(jnp.int32)
    pad = E_PAD - E
    srcs = jnp.concatenate([ei[0], jnp.zeros((pad,), jnp.int32)])
    srcs2 = jnp.concatenate([srcs, srcs + N])
    dsts = jnp.concatenate([ei[1], jnp.full((pad,), N, jnp.int32)])
    dsts2 = dsts.reshape(E_PAD // C, C)
    rowidx2 = jnp.arange(ACC_ROWS, dtype=jnp.int32).reshape(NT, ZROWS // C, C)
    zrow = jnp.zeros((C, H), jnp.float32)
    ones = jnp.ones((C, H), jnp.float32)
    b1r = b1.reshape(1, D)
    b2r = b2.reshape(1, D)

    y1 = _mm(x, W1)
    a1f, degf = _seg_mean(True)(y1.reshape(2 * N, H), srcs2, dsts2,
                                rowidx2, zrow, ones)
    a1 = a1f.reshape(2, ACC_ROWS, H)
    deg = degf.reshape(2, ACC_ROWS, H)
    y2 = _mid(y1, a1, deg, b1r, W2)
    a2f, = _seg_mean(False)(y2.reshape(2 * N, H), srcs2, dsts2,
                            rowidx2, zrow, ones)
    a2 = a2f.reshape(2, ACC_ROWS, H)
    return _fin(y2, a2, deg, b2r)


# speedup vs baseline: 3.5007x; 1.0298x over previous
import functools

import jax
import jax.numpy as jnp
from jax import lax
from jax.experimental import pallas as pl
from jax.experimental.pallas import tpu as pltpu
from jax.experimental.pallas import tpu_sc as plsc

N = 10000
E = 160000
D = 256
H = 128
NT = 16
C = 128
NCH = 80
E_PAD = NT * C * NCH
ACC_ROWS = 10240
ZROWS = 640
BB = 8
RBLK = 400
GRID = N // RBLK


def _seg_mean(with_deg):
    outs = [jax.ShapeDtypeStruct((2 * ACC_ROWS, H), jnp.float32)]
    if with_deg:
        outs.append(jax.ShapeDtypeStruct((2 * ACC_ROWS, H), jnp.float32))
    scratch = [
        pltpu.VMEM((ZROWS // C, C), jnp.int32),
        pltpu.VMEM((BB * C,), jnp.int32),
        pltpu.VMEM((NCH, C), jnp.int32),
        pltpu.VMEM((C, H), jnp.float32),
        pltpu.VMEM((C, H), jnp.float32),
        pltpu.VMEM_SHARED((ACC_ROWS, H), jnp.float32),
        pltpu.SemaphoreType.DMA,
        pltpu.SemaphoreType.DMA,
    ]
    mesh = plsc.VectorSubcoreMesh(core_axis_name="c", subcore_axis_name="s",
                                  num_cores=2, num_subcores=NT)

    @functools.partial(pl.kernel, out_type=outs, mesh=mesh,
                       scratch_types=scratch)
    def k(y2n, srcs2, dsts2, rowidx2, zrow, ones_in, *rest):
        if with_deg:
            (agg_out, deg_out, idx_v, src_b, dst_b,
             rows_v0, rows_v1, acc, sem0, sem1) = rest
        else:
            (agg_out, idx_v, src_b, dst_b,
             rows_v0, rows_v1, acc, sem0, sem1) = rest
        rows = (rows_v0, rows_v1)
        sems = (sem0, sem1)
        c = lax.axis_index("c")
        s = lax.axis_index("s")
        nslab = ZROWS // C

        def load_slab_idx():
            pltpu.sync_copy(rowidx2.at[s], idx_v)

        def zero_slab():
            pltpu.sync_copy(zrow, rows_v0)
            for j in range(nslab):
                pltpu.sync_copy(rows_v0, acc.at[idx_v.at[j]])

        def read_slab(out_ref):
            g = pltpu.async_copy(acc.at[idx_v.at[0]], rows_v0, sem0)
            for j in range(nslab):
                if j + 1 < nslab:
                    gn = pltpu.async_copy(acc.at[idx_v.at[j + 1]],
                                          rows[(j + 1) % 2], sems[(j + 1) % 2])
                g.wait()
                orows = pl.ds(c * ACC_ROWS + s * ZROWS + j * C, C)
                pltpu.sync_copy(rows[j % 2], out_ref.at[orows])
                if j + 1 < nslab:
                    g = gn

        load_slab_idx()
        pltpu.sync_copy(dsts2.at[pl.ds(s * NCH, NCH)], dst_b)
        if with_deg:
            zero_slab()
            pltpu.sync_copy(ones_in, rows_v1)
            plsc.subcore_barrier()
            descs = []
            for j in range(NCH):
                descs.append(pltpu.async_copy(rows_v1, acc.at[dst_b.at[j]],
                                              sem0, add=True))
                if j >= 8:
                    descs[j - 8].wait()
            for d in descs[-8:]:
                d.wait()
            plsc.subcore_barrier()
            read_slab(deg_out)
            plsc.subcore_barrier()

        zero_slab()
        plsc.subcore_barrier()

        def step(t, carry):
            base = c * E_PAD + (s * NCH + t * BB) * C
            pltpu.sync_copy(srcs2.at[pl.ds(base, BB * C)], src_b)
            g = pltpu.async_copy(y2n.at[src_b.at[pl.ds(0, C)]],
                                 rows_v0, sem0)
            for j in range(BB):
                if j + 1 < BB:
                    gn = pltpu.async_copy(
                        y2n.at[src_b.at[pl.ds((j + 1) * C, C)]],
                        rows[(j + 1) % 2], sems[(j + 1) % 2])
                g.wait()
                pltpu.sync_copy(rows[j % 2], acc.at[dst_b.at[t * BB + j]],
                                add=True)
                if j + 1 < BB:
                    g = gn
            return carry

        lax.fori_loop(0, NCH // BB, step, 0)
        plsc.subcore_barrier()
        read_slab(agg_out)

    return k


def _mm(x, w):
    def body(x_ref, w_ref, o_ref):
        r = jnp.dot(x_ref[...], w_ref[...], preferred_element_type=jnp.float32)
        o_ref[0] = r[:, :H]
        o_ref[1] = r[:, H:]

    return pl.pallas_call(
        body, grid=(GRID,),
        in_specs=[pl.BlockSpec((RBLK, D), lambda i: (i, 0)),
                  pl.BlockSpec((D, D), lambda i: (0, 0))],
        out_specs=pl.BlockSpec((2, RBLK, H), lambda i: (0, i, 0)),
        out_shape=jax.ShapeDtypeStruct((2, N, H), jnp.float32),
    )(x, w)


def _mid(y_st, agg_st, deg_st, b, w):
    def body(y_ref, a_ref, deg_ref, b_ref, w_ref, o_ref):
        inv = 1.0 / jnp.maximum(deg_ref[0][:, 0:1], 1.0)
        xlo = y_ref[0] + a_ref[0] * inv
        xhi = y_ref[1] + a_ref[1] * inv
        xx = jnp.concatenate([xlo, xhi], axis=1) + b_ref[...]
        xx = jnp.maximum(xx, 0.0)
        r = jnp.dot(xx, w_ref[...], preferred_element_type=jnp.float32)
        o_ref[0] = r[:, :H]
        o_ref[1] = r[:, H:]

    return pl.pallas_call(
        body, grid=(GRID,),
        in_specs=[pl.BlockSpec((2, RBLK, H), lambda i: (0, i, 0)),
                  pl.BlockSpec((2, RBLK, H), lambda i: (0, i, 0)),
                  pl.BlockSpec((1, RBLK, H), lambda i: (0, i, 0)),
                  pl.BlockSpec((1, D), lambda i: (0, 0)),
                  pl.BlockSpec((D, D), lambda i: (0, 0))],
        out_specs=pl.BlockSpec((2, RBLK, H), lambda i: (0, i, 0)),
        out_shape=jax.ShapeDtypeStruct((2, N, H), jnp.float32),
    )(y_st, agg_st, deg_st, b, w)


def _fin(y_st, agg_st, deg_st, b):
    def body(y_ref, a_ref, deg_ref, b_ref, o_ref):
        inv = 1.0 / jnp.maximum(deg_ref[0][:, 0:1], 1.0)
        xlo = y_ref[0] + a_ref[0] * inv
        xhi = y_ref[1] + a_ref[1] * inv
        o_ref[...] = jnp.concatenate([xlo, xhi], axis=1) + b_ref[...]

    return pl.pallas_call(
        body, grid=(GRID,),
        in_specs=[pl.BlockSpec((2, RBLK, H), lambda i: (0, i, 0)),
                  pl.BlockSpec((2, RBLK, H), lambda i: (0, i, 0)),
                  pl.BlockSpec((1, RBLK, H), lambda i: (0, i, 0)),
                  pl.BlockSpec((1, D), lambda i: (0, 0))],
        out_specs=pl.BlockSpec((RBLK, D), lambda i: (i, 0)),
        out_shape=jax.ShapeDtypeStruct((N, D), jnp.float32),
    )(y_st, agg_st, deg_st, b)


def kernel(features, edge_index, W1, b1, W2, b2):
    x = features.astype(jnp.float32)
    ei = edge_index.astype(jnp.int32)
    pad = E_PAD - E
    srcs = jnp.concatenate([ei[0], jnp.zeros((pad,), jnp.int32)])
    srcs2 = jnp.concatenate([srcs, srcs + N])
    dsts = jnp.concatenate([ei[1], jnp.full((pad,), N, jnp.int32)])
    dsts2 = dsts.reshape(E_PAD // C, C)
    rowidx2 = jnp.arange(ACC_ROWS, dtype=jnp.int32).reshape(NT, ZROWS // C, C)
    zrow = jnp.zeros((C, H), jnp.float32)
    ones = jnp.ones((C, H), jnp.float32)
    b1r = b1.reshape(1, D)
    b2r = b2.reshape(1, D)

    y1 = _mm(x, W1)
    a1f, degf = _seg_mean(True)(y1.reshape(2 * N, H), srcs2, dsts2,
                                rowidx2, zrow, ones)
    a1 = a1f.reshape(2, ACC_ROWS, H)
    deg = degf.reshape(2, ACC_ROWS, H)
    y2 = _mid(y1, a1, deg, b1r, W2)
    a2f, = _seg_mean(False)(y2.reshape(2 * N, H), srcs2, dsts2,
                            rowidx2, zrow, ones)
    a2 = a2f.reshape(2, ACC_ROWS, H)
    return _fin(y2, a2, deg, b2r)


# speedup vs baseline: 3.5055x; 1.0014x over previous
import functools

import jax
import jax.numpy as jnp
from jax import lax
from jax.experimental import pallas as pl
from jax.experimental.pallas import tpu as pltpu
from jax.experimental.pallas import tpu_sc as plsc

N = 10000
E = 160000
D = 256
H = 128
NT = 16
C = 128
NCH = 80
E_PAD = NT * C * NCH
ACC_ROWS = 10240
ZROWS = 640
BB = 8
RBLK = 400
GRID = N // RBLK


def _seg_mean(with_deg):
    outs = [jax.ShapeDtypeStruct((2 * ACC_ROWS, H), jnp.float32)]
    if with_deg:
        outs.append(jax.ShapeDtypeStruct((2 * ACC_ROWS, H), jnp.float32))
    scratch = [
        pltpu.VMEM((ZROWS // C, C), jnp.int32),
        pltpu.VMEM((BB * C,), jnp.int32),
        pltpu.VMEM((NCH, C), jnp.int32),
        pltpu.VMEM((C, H), jnp.float32),
        pltpu.VMEM((C, H), jnp.float32),
        pltpu.VMEM_SHARED((ACC_ROWS, H), jnp.float32),
        pltpu.SemaphoreType.DMA,
        pltpu.SemaphoreType.DMA,
        pltpu.SemaphoreType.DMA,
        pltpu.SemaphoreType.DMA,
    ]
    mesh = plsc.VectorSubcoreMesh(core_axis_name="c", subcore_axis_name="s",
                                  num_cores=2, num_subcores=NT)

    @functools.partial(pl.kernel, out_type=outs, mesh=mesh,
                       scratch_types=scratch)
    def k(y2n, srcs2, dsts2, rowidx2, zrow, ones_in, *rest):
        if with_deg:
            (agg_out, deg_out, idx_v, src_b, dst_b,
             rows_v0, rows_v1, acc, sem0, sem1, sem2, sem3) = rest
        else:
            (agg_out, idx_v, src_b, dst_b,
             rows_v0, rows_v1, acc, sem0, sem1, sem2, sem3) = rest
        rows = (rows_v0, rows_v1)
        sems = (sem0, sem1)
        c = lax.axis_index("c")
        s = lax.axis_index("s")
        nslab = ZROWS // C

        def load_slab_idx():
            pltpu.sync_copy(rowidx2.at[s], idx_v)

        def zero_slab():
            pltpu.sync_copy(zrow, rows_v0)
            for j in range(nslab):
                pltpu.sync_copy(rows_v0, acc.at[idx_v.at[j]])

        def read_slab(out_ref):
            g = pltpu.async_copy(acc.at[idx_v.at[0]], rows_v0, sem0)
            for j in range(nslab):
                if j + 1 < nslab:
                    gn = pltpu.async_copy(acc.at[idx_v.at[j + 1]],
                                          rows[(j + 1) % 2], sems[(j + 1) % 2])
                g.wait()
                orows = pl.ds(c * ACC_ROWS + s * ZROWS + j * C, C)
                pltpu.sync_copy(rows[j % 2], out_ref.at[orows])
                if j + 1 < nslab:
                    g = gn

        load_slab_idx()
        pltpu.sync_copy(dsts2.at[pl.ds(s * NCH, NCH)], dst_b)
        if with_deg:
            zero_slab()
            pltpu.sync_copy(ones_in, rows_v1)
            plsc.subcore_barrier()
            descs = []
            for j in range(NCH):
                descs.append(pltpu.async_copy(rows_v1, acc.at[dst_b.at[j]],
                                              sem0, add=True))
                if j >= 8:
                    descs[j - 8].wait()
            for d in descs[-8:]:
                d.wait()
            plsc.subcore_barrier()
            read_slab(deg_out)
            plsc.subcore_barrier()

        zero_slab()
        plsc.subcore_barrier()

        scsems = (sem2, sem3)

        def step(t, carry):
            base = c * E_PAD + (s * NCH + t * BB) * C
            pltpu.sync_copy(srcs2.at[pl.ds(base, BB * C)], src_b)
            g = pltpu.async_copy(y2n.at[src_b.at[pl.ds(0, C)]],
                                 rows_v0, sem0)
            scd = [None, None]
            for j in range(BB):
                slot = j % 2
                if j + 1 < BB:
                    nslot = (j + 1) % 2
                    if scd[nslot] is not None:
                        scd[nslot].wait()
                        scd[nslot] = None
                    gn = pltpu.async_copy(
                        y2n.at[src_b.at[pl.ds((j + 1) * C, C)]],
                        rows[nslot], sems[nslot])
                g.wait()
                scd[slot] = pltpu.async_copy(
                    rows[slot], acc.at[dst_b.at[t * BB + j]],
                    scsems[slot], add=True)
                if j + 1 < BB:
                    g = gn
            for dsc in scd:
                if dsc is not None:
                    dsc.wait()
            return carry

        lax.fori_loop(0, NCH // BB, step, 0)
        plsc.subcore_barrier()
        read_slab(agg_out)

    return k


def _mm(x, w):
    def body(x_ref, w_ref, o_ref):
        r = jnp.dot(x_ref[...], w_ref[...], preferred_element_type=jnp.float32)
        o_ref[0] = r[:, :H]
        o_ref[1] = r[:, H:]

    return pl.pallas_call(
        body, grid=(GRID,),
        in_specs=[pl.BlockSpec((RBLK, D), lambda i: (i, 0)),
                  pl.BlockSpec((D, D), lambda i: (0, 0))],
        out_specs=pl.BlockSpec((2, RBLK, H), lambda i: (0, i, 0)),
        out_shape=jax.ShapeDtypeStruct((2, N, H), jnp.float32),
    )(x, w)


def _mid(y_st, agg_st, deg_st, b, w):
    def body(y_ref, a_ref, deg_ref, b_ref, w_ref, o_ref):
        inv = 1.0 / jnp.maximum(deg_ref[0][:, 0:1], 1.0)
        xlo = y_ref[0] + a_ref[0] * inv
        xhi = y_ref[1] + a_ref[1] * inv
        xx = jnp.concatenate([xlo, xhi], axis=1) + b_ref[...]
        xx = jnp.maximum(xx, 0.0)
        r = jnp.dot(xx, w_ref[...], preferred_element_type=jnp.float32)
        o_ref[0] = r[:, :H]
        o_ref[1] = r[:, H:]

    return pl.pallas_call(
        body, grid=(GRID,),
        in_specs=[pl.BlockSpec((2, RBLK, H), lambda i: (0, i, 0)),
                  pl.BlockSpec((2, RBLK, H), lambda i: (0, i, 0)),
                  pl.BlockSpec((1, RBLK, H), lambda i: (0, i, 0)),
                  pl.BlockSpec((1, D), lambda i: (0, 0)),
                  pl.BlockSpec((D, D), lambda i: (0, 0))],
        out_specs=pl.BlockSpec((2, RBLK, H), lambda i: (0, i, 0)),
        out_shape=jax.ShapeDtypeStruct((2, N, H), jnp.float32),
    )(y_st, agg_st, deg_st, b, w)


def _fin(y_st, agg_st, deg_st, b):
    def body(y_ref, a_ref, deg_ref, b_ref, o_ref):
        inv = 1.0 / jnp.maximum(deg_ref[0][:, 0:1], 1.0)
        xlo = y_ref[0] + a_ref[0] * inv
        xhi = y_ref[1] + a_ref[1] * inv
        o_ref[...] = jnp.concatenate([xlo, xhi], axis=1) + b_ref[...]

    return pl.pallas_call(
        body, grid=(GRID,),
        in_specs=[pl.BlockSpec((2, RBLK, H), lambda i: (0, i, 0)),
                  pl.BlockSpec((2, RBLK, H), lambda i: (0, i, 0)),
                  pl.BlockSpec((1, RBLK, H), lambda i: (0, i, 0)),
                  pl.BlockSpec((1, D), lambda i: (0, 0))],
        out_specs=pl.BlockSpec((RBLK, D), lambda i: (i, 0)),
        out_shape=jax.ShapeDtypeStruct((N, D), jnp.float32),
    )(y_st, agg_st, deg_st, b)


def kernel(features, edge_index, W1, b1, W2, b2):
    x = features.astype(jnp.float32)
    ei = edge_index.astype(jnp.int32)
    pad = E_PAD - E
    srcs = jnp.concatenate([ei[0], jnp.zeros((pad,), jnp.int32)])
    srcs2 = jnp.concatenate([srcs, srcs + N])
    dsts = jnp.concatenate([ei[1], jnp.full((pad,), N, jnp.int32)])
    dsts2 = dsts.reshape(E_PAD // C, C)
    rowidx2 = jnp.arange(ACC_ROWS, dtype=jnp.int32).reshape(NT, ZROWS // C, C)
    zrow = jnp.zeros((C, H), jnp.float32)
    ones = jnp.ones((C, H), jnp.float32)
    b1r = b1.reshape(1, D)
    b2r = b2.reshape(1, D)

    y1 = _mm(x, W1)
    a1f, degf = _seg_mean(True)(y1.reshape(2 * N, H), srcs2, dsts2,
                                rowidx2, zrow, ones)
    a1 = a1f.reshape(2, ACC_ROWS, H)
    deg = degf.reshape(2, ACC_ROWS, H)
    y2 = _mid(y1, a1, deg, b1r, W2)
    a2f, = _seg_mean(False)(y2.reshape(2 * N, H), srcs2, dsts2,
                            rowidx2, zrow, ones)
    a2 = a2f.reshape(2, ACC_ROWS, H)
    return _fin(y2, a2, deg, b2r)
